# Initial kernel scaffold; baseline (speedup 1.0000x reference)
import jax
import jax.numpy as jnp
from jax.experimental import pallas as pl


def kernel(x, edge_index, W, b, gamma, beta):
    raise NotImplementedError("write your pallas kernel here")


# speedup vs baseline: 13.6894x; 13.6894x over previous
import functools

import jax
import jax.numpy as jnp
from jax import lax
from jax.experimental import pallas as pl
from jax.experimental.pallas import tpu as pltpu
import jax.experimental.pallas.tpu_sc as plsc

N_NODES = 10000
N_EDGES = 320000
D = 128
NC = 2
NS = 16
NW = NC * NS
CHUNK = 128
STRIPE = 640
NPAD = STRIPE * 16
BASE_CHUNKS = 80
NCHUNKS = BASE_CHUNKS * NW
E_PAD = NCHUNKS * CHUNK
DUMMY_DST = NPAD - 8
SUP = 8


def _mesh():
    return plsc.VectorSubcoreMesh(
        core_axis_name="c", subcore_axis_name="s", num_cores=NC, num_subcores=NS
    )


def _worker_id():
    return lax.axis_index("s") + NS * lax.axis_index("c")


def _deg_body(dst2_hbm, zeros_hbm, out_hbm, idx_v, ones_v, acc):
    cid = lax.axis_index("c")
    sid = lax.axis_index("s")
    wid = _worker_id()

    pltpu.sync_copy(zeros_hbm, acc.at[pl.ds(sid * STRIPE, STRIPE)])
    for j in range(CHUNK // 16):
        ones_v[pl.ds(j * 16, 16)] = jnp.full((16,), 1.0, jnp.float32)
    plsc.subcore_barrier()

    pltpu.sync_copy(dst2_hbm.at[pl.ds(wid * BASE_CHUNKS, BASE_CHUNKS)], idx_v)

    def body(j, _):
        pltpu.sync_copy(ones_v, acc.at[idx_v.at[j]], add=True)
        return ()

    lax.fori_loop(0, BASE_CHUNKS, body, (), unroll=False)

    plsc.subcore_barrier()
    pltpu.sync_copy(
        acc.at[pl.ds(sid * STRIPE, STRIPE)],
        out_hbm.at[cid, pl.ds(sid * STRIPE, STRIPE)],
    )


def _deg_call(dst2, zeros1d):
    return pl.kernel(
        _deg_body,
        out_type=jax.ShapeDtypeStruct((NC, NPAD), jnp.float32),
        mesh=_mesh(),
        scratch_types=[
            pltpu.VMEM((BASE_CHUNKS, CHUNK), jnp.int32),
            pltpu.VMEM((CHUNK,), jnp.float32),
            pltpu.VMEM_SHARED((NPAD,), jnp.float32),
        ],
    )(dst2, zeros1d)


def _mm_body(x_ref, w_ref, degp_ref, h2_ref, dinv_ref):
    deg = degp_ref[0, :N_NODES] + degp_ref[1, :N_NODES] + 1.0
    dinv = lax.rsqrt(deg)
    dinv_ref[...] = dinv
    h = jnp.dot(x_ref[...], w_ref[...], preferred_element_type=jnp.float32)
    h2_ref[...] = h * dinv[:, None]


def _mm_call(x, w, degp):
    return pl.pallas_call(
        _mm_body,
        out_shape=(
            jax.ShapeDtypeStruct((N_NODES, D), jnp.float32),
            jax.ShapeDtypeStruct((N_NODES,), jnp.float32),
        ),
    )(x, w, degp)


def _agg_body(src2_hbm, dst2_hbm, h2_hbm, zeros_hbm, out_hbm,
              isrc_v, idst_v, buf0, buf1, sem0, sem1, acc):
    cid = lax.axis_index("c")
    sid = lax.axis_index("s")
    wid = _worker_id()

    pltpu.sync_copy(zeros_hbm, acc.at[pl.ds(sid * STRIPE, STRIPE)])
    plsc.subcore_barrier()

    def outer(t, _):
        base = wid * BASE_CHUNKS + t * SUP
        pltpu.sync_copy(src2_hbm.at[pl.ds(base, SUP)], isrc_v)
        pltpu.sync_copy(dst2_hbm.at[pl.ds(base, SUP)], idst_v)
        pltpu.async_copy(h2_hbm.at[isrc_v.at[0]], buf0, sem0)

        def body(k, _):
            j0 = 2 * k
            pltpu.async_copy(h2_hbm.at[isrc_v.at[j0 + 1]], buf1, sem1)
            pltpu.make_async_copy(h2_hbm.at[isrc_v.at[j0]], buf0, sem0).wait()
            pltpu.sync_copy(buf0, acc.at[idst_v.at[j0]], add=True)

            @pl.when(j0 + 2 < SUP)
            def _():
                pltpu.async_copy(h2_hbm.at[isrc_v.at[j0 + 2]], buf0, sem0)

            pltpu.make_async_copy(h2_hbm.at[isrc_v.at[j0 + 1]], buf1, sem1).wait()
            pltpu.sync_copy(buf1, acc.at[idst_v.at[j0 + 1]], add=True)
            return ()

        lax.fori_loop(0, SUP // 2, body, (), unroll=False)
        return ()

    lax.fori_loop(0, BASE_CHUNKS // SUP, outer, (), unroll=False)

    plsc.subcore_barrier()
    pltpu.sync_copy(
        acc.at[pl.ds(sid * STRIPE, STRIPE)],
        out_hbm.at[cid, pl.ds(sid * STRIPE, STRIPE)],
    )


def _agg_call(src2, dst2, h2, zeros2d):
    return pl.kernel(
        _agg_body,
        out_type=jax.ShapeDtypeStruct((NC, NPAD, D), jnp.float32),
        mesh=_mesh(),
        scratch_types=[
            pltpu.VMEM((SUP, CHUNK), jnp.int32),
            pltpu.VMEM((SUP, CHUNK), jnp.int32),
            pltpu.VMEM((CHUNK, D), jnp.float32),
            pltpu.VMEM((CHUNK, D), jnp.float32),
            pltpu.SemaphoreType.DMA,
            pltpu.SemaphoreType.DMA,
            pltpu.VMEM_SHARED((NPAD, D), jnp.float32),
        ],
    )(src2, dst2, h2, zeros2d)


def _bn_body(aggp_ref, h2_ref, dinv_ref, b_ref, gamma_ref, beta_ref, out_ref):
    s = aggp_ref[0, :N_NODES, :] + aggp_ref[1, :N_NODES, :] + h2_ref[...]
    pre = s * dinv_ref[...][:, None] + b_ref[...]
    mean = jnp.mean(pre, axis=0)
    var = jnp.mean((pre - mean[None, :]) ** 2, axis=0)
    scaled = (pre - mean[None, :]) * lax.rsqrt(var + 1e-5)[None, :]
    out_ref[...] = jnp.maximum(scaled * gamma_ref[...] + beta_ref[...], 0.0)


def _bn_call(aggp, h2, dinv, b, gamma, beta):
    return pl.pallas_call(
        _bn_body,
        out_shape=jax.ShapeDtypeStruct((N_NODES, D), jnp.float32),
    )(aggp, h2, dinv, b, gamma, beta)


def kernel(x, edge_index, W, b, gamma, beta):
    ei = edge_index.astype(jnp.int32)
    npad = E_PAD - N_EDGES
    src2 = jnp.concatenate(
        [ei[0], jnp.zeros((npad,), jnp.int32)]).reshape(NCHUNKS, CHUNK)
    dst2 = jnp.concatenate(
        [ei[1], jnp.full((npad,), DUMMY_DST, jnp.int32)]).reshape(NCHUNKS, CHUNK)
    zeros1d = jnp.zeros((STRIPE,), jnp.float32)
    zeros2d = jnp.zeros((STRIPE, D), jnp.float32)

    degp = _deg_call(dst2, zeros1d)
    h2, dinv = _mm_call(x, W, degp)
    aggp = _agg_call(src2, dst2, h2, zeros2d)
    return _bn_call(aggp, h2, dinv, b, gamma, beta)


# speedup vs baseline: 13.7296x; 1.0029x over previous
import functools

import jax
import jax.numpy as jnp
from jax import lax
from jax.experimental import pallas as pl
from jax.experimental.pallas import tpu as pltpu
import jax.experimental.pallas.tpu_sc as plsc

N_NODES = 10000
N_EDGES = 320000
D = 128
NC = 2
NS = 16
NW = NC * NS
CHUNK = 128
STRIPE = 640
NPAD = STRIPE * 16
BASE_CHUNKS = 80
NCHUNKS = BASE_CHUNKS * NW
E_PAD = NCHUNKS * CHUNK
DUMMY_DST = NPAD - 8
SUP = 8


def _mesh():
    return plsc.VectorSubcoreMesh(
        core_axis_name="c", subcore_axis_name="s", num_cores=NC, num_subcores=NS
    )


def _worker_id():
    return lax.axis_index("s") + NS * lax.axis_index("c")


def _deg_body(dst2_hbm, zeros_hbm, out_hbm, idx_v, ones_v, acc):
    cid = lax.axis_index("c")
    sid = lax.axis_index("s")
    wid = _worker_id()

    pltpu.sync_copy(zeros_hbm, acc.at[pl.ds(sid * STRIPE, STRIPE)])
    for j in range(CHUNK // 16):
        ones_v[pl.ds(j * 16, 16)] = jnp.full((16,), 1.0, jnp.float32)
    plsc.subcore_barrier()

    pltpu.sync_copy(dst2_hbm.at[pl.ds(wid * BASE_CHUNKS, BASE_CHUNKS)], idx_v)

    def body(j, _):
        pltpu.sync_copy(ones_v, acc.at[idx_v.at[j]], add=True)
        return ()

    lax.fori_loop(0, BASE_CHUNKS, body, (), unroll=False)

    plsc.subcore_barrier()
    pltpu.sync_copy(
        acc.at[pl.ds(sid * STRIPE, STRIPE)],
        out_hbm.at[cid, pl.ds(sid * STRIPE, STRIPE)],
    )


def _deg_call(dst2, zeros1d):
    return pl.kernel(
        _deg_body,
        out_type=jax.ShapeDtypeStruct((NC, NPAD), jnp.float32),
        mesh=_mesh(),
        scratch_types=[
            pltpu.VMEM((BASE_CHUNKS, CHUNK), jnp.int32),
            pltpu.VMEM((CHUNK,), jnp.float32),
            pltpu.VMEM_SHARED((NPAD,), jnp.float32),
        ],
    )(dst2, zeros1d)


def _mm_body(x_ref, w_ref, degp_ref, h2_ref, dinv_ref):
    deg = degp_ref[0, :N_NODES] + degp_ref[1, :N_NODES] + 1.0
    dinv = lax.rsqrt(deg)
    dinv_ref[...] = dinv
    h = jnp.dot(x_ref[...], w_ref[...], preferred_element_type=jnp.float32)
    h2_ref[...] = h * dinv[:, None]


def _mm_call(x, w, degp):
    return pl.pallas_call(
        _mm_body,
        out_shape=(
            jax.ShapeDtypeStruct((N_NODES, D), jnp.float32),
            jax.ShapeDtypeStruct((N_NODES,), jnp.float32),
        ),
    )(x, w, degp)


def _agg_body(src2_hbm, dst2_hbm, h2_hbm, zeros_hbm, out_hbm,
              isrc_v, idst_v, buf0, buf1, sem0, sem1, acc):
    cid = lax.axis_index("c")
    sid = lax.axis_index("s")
    wid = _worker_id()

    pltpu.sync_copy(zeros_hbm, acc.at[pl.ds(sid * STRIPE, STRIPE)])
    plsc.subcore_barrier()

    def outer(t, _):
        base = wid * BASE_CHUNKS + t * SUP
        pltpu.sync_copy(src2_hbm.at[pl.ds(base, SUP)], isrc_v)
        pltpu.sync_copy(dst2_hbm.at[pl.ds(base, SUP)], idst_v)
        pltpu.async_copy(h2_hbm.at[isrc_v.at[0]], buf0, sem0)

        def body(k, _):
            j0 = 2 * k
            pltpu.async_copy(h2_hbm.at[isrc_v.at[j0 + 1]], buf1, sem1)
            pltpu.make_async_copy(h2_hbm.at[isrc_v.at[j0]], buf0, sem0).wait()
            pltpu.sync_copy(buf0, acc.at[idst_v.at[j0]], add=True)

            @pl.when(j0 + 2 < SUP)
            def _():
                pltpu.async_copy(h2_hbm.at[isrc_v.at[j0 + 2]], buf0, sem0)

            pltpu.make_async_copy(h2_hbm.at[isrc_v.at[j0 + 1]], buf1, sem1).wait()
            pltpu.sync_copy(buf1, acc.at[idst_v.at[j0 + 1]], add=True)
            return ()

        lax.fori_loop(0, SUP // 2, body, (), unroll=False)
        return ()

    lax.fori_loop(0, BASE_CHUNKS // SUP, outer, (), unroll=False)

    plsc.subcore_barrier()
    pltpu.sync_copy(
        acc.at[pl.ds(sid * STRIPE, STRIPE)],
        out_hbm.at[cid, pl.ds(sid * STRIPE, STRIPE)],
    )


def _agg_call(src2, dst2, h2, zeros2d):
    return pl.kernel(
        _agg_body,
        out_type=jax.ShapeDtypeStruct((NC, NPAD, D), jnp.float32),
        mesh=_mesh(),
        scratch_types=[
            pltpu.VMEM((SUP, CHUNK), jnp.int32),
            pltpu.VMEM((SUP, CHUNK), jnp.int32),
            pltpu.VMEM((CHUNK, D), jnp.float32),
            pltpu.VMEM((CHUNK, D), jnp.float32),
            pltpu.SemaphoreType.DMA,
            pltpu.SemaphoreType.DMA,
            pltpu.VMEM_SHARED((NPAD, D), jnp.float32),
        ],
    )(src2, dst2, h2, zeros2d)


def _bn_body(aggp_ref, h2_ref, dinv_ref, b_ref, gamma_ref, beta_ref, out_ref):
    s = aggp_ref[0, :N_NODES, :] + aggp_ref[1, :N_NODES, :] + h2_ref[...]
    pre = s * dinv_ref[...][:, None] + b_ref[...]
    mean = jnp.mean(pre, axis=0)
    var = jnp.mean((pre - mean[None, :]) ** 2, axis=0)
    scaled = (pre - mean[None, :]) * lax.rsqrt(var + 1e-5)[None, :]
    out_ref[...] = jnp.maximum(scaled * gamma_ref[...] + beta_ref[...], 0.0)


def _bn_call(aggp, h2, dinv, b, gamma, beta):
    return pl.pallas_call(
        _bn_body,
        out_shape=jax.ShapeDtypeStruct((N_NODES, D), jnp.float32),
    )(aggp, h2, dinv, b, gamma, beta)


def kernel(x, edge_index, W, b, gamma, beta):
    ei = edge_index.astype(jnp.int32)
    npad = E_PAD - N_EDGES
    src2 = jnp.concatenate(
        [ei[0], jnp.zeros((npad,), jnp.int32)]).reshape(NCHUNKS, CHUNK)
    pad_dst = N_NODES + jnp.arange(npad, dtype=jnp.int32) % (NPAD - N_NODES)
    dst2 = jnp.concatenate([ei[1], pad_dst]).reshape(NCHUNKS, CHUNK)
    zeros1d = jnp.zeros((STRIPE,), jnp.float32)
    zeros2d = jnp.zeros((STRIPE, D), jnp.float32)

    degp = _deg_call(dst2, zeros1d)
    h2, dinv = _mm_call(x, W, degp)
    aggp = _agg_call(src2, dst2, h2, zeros2d)
    return _bn_call(aggp, h2, dinv, b, gamma, beta)


# speedup vs baseline: 39.8674x; 2.9038x over previous
import functools

import jax
import jax.numpy as jnp
from jax import lax
from jax.experimental import pallas as pl
from jax.experimental.pallas import tpu as pltpu
import jax.experimental.pallas.tpu_sc as plsc

N_NODES = 10000
N_EDGES = 320000
D = 128
NC = 2
NS = 16
NW = NC * NS
CHUNK = 128
STRIPE = 640
NPAD = STRIPE * 16
BASE_CHUNKS = 80
NCHUNKS = BASE_CHUNKS * NW
E_PAD = NCHUNKS * CHUNK
DUMMY_DST = NPAD - 8
SUP = 8


def _mesh():
    return plsc.VectorSubcoreMesh(
        core_axis_name="c", subcore_axis_name="s", num_cores=NC, num_subcores=NS
    )


def _worker_id():
    return lax.axis_index("s") + NS * lax.axis_index("c")


def _deg_body(dst2_hbm, zeros_hbm, out_hbm, idx_v, ones_v, acc):
    cid = lax.axis_index("c")
    sid = lax.axis_index("s")
    wid = _worker_id()

    pltpu.sync_copy(zeros_hbm, acc.at[pl.ds(sid * STRIPE, STRIPE)])
    for j in range(CHUNK // 16):
        ones_v[pl.ds(j * 16, 16)] = jnp.full((16,), 1.0, jnp.float32)
    plsc.subcore_barrier()

    pltpu.sync_copy(dst2_hbm.at[pl.ds(wid * BASE_CHUNKS, BASE_CHUNKS)], idx_v)

    def body(j, _):
        pltpu.sync_copy(ones_v, acc.at[idx_v.at[j]], add=True)
        return ()

    lax.fori_loop(0, BASE_CHUNKS, body, (), unroll=False)

    plsc.subcore_barrier()
    pltpu.sync_copy(
        acc.at[pl.ds(sid * STRIPE, STRIPE)],
        out_hbm.at[cid, pl.ds(sid * STRIPE, STRIPE)],
    )


def _deg_call(dst2, zeros1d):
    return pl.kernel(
        _deg_body,
        out_type=jax.ShapeDtypeStruct((NC, NPAD), jnp.float32),
        mesh=_mesh(),
        scratch_types=[
            pltpu.VMEM((BASE_CHUNKS, CHUNK), jnp.int32),
            pltpu.VMEM((CHUNK,), jnp.float32),
            pltpu.VMEM_SHARED((NPAD,), jnp.float32),
        ],
    )(dst2, zeros1d)


def _mm_body(x_ref, w_ref, degp_ref, h2_ref, dinv_ref):
    deg = degp_ref[0, :N_NODES] + degp_ref[1, :N_NODES] + 1.0
    dinv = lax.rsqrt(deg)
    dinv_ref[...] = dinv
    h = jnp.dot(x_ref[...], w_ref[...], preferred_element_type=jnp.float32)
    h2_ref[...] = h * dinv[:, None]


def _mm_call(x, w, degp):
    return pl.pallas_call(
        _mm_body,
        out_shape=(
            jax.ShapeDtypeStruct((N_NODES, D), jnp.float32),
            jax.ShapeDtypeStruct((N_NODES,), jnp.float32),
        ),
    )(x, w, degp)


def _agg_body(src2_hbm, dst2_hbm, h2_hbm, zeros_hbm, out_hbm,
              isrc_v, idst_v, buf0, buf1, sem0, sem1, acc):
    cid = lax.axis_index("c")
    sid = lax.axis_index("s")
    wid = _worker_id()

    pltpu.sync_copy(zeros_hbm, acc.at[pl.ds(sid * STRIPE, STRIPE)])
    plsc.subcore_barrier()

    def outer(t, _):
        base = wid * BASE_CHUNKS + t * SUP
        pltpu.sync_copy(src2_hbm.at[pl.ds(base, SUP)], isrc_v)
        pltpu.sync_copy(dst2_hbm.at[pl.ds(base, SUP)], idst_v)
        pltpu.async_copy(h2_hbm.at[isrc_v.at[0]], buf0, sem0)

        def body(k, _):
            j0 = 2 * k
            pltpu.async_copy(h2_hbm.at[isrc_v.at[j0 + 1]], buf1, sem1)
            pltpu.make_async_copy(h2_hbm.at[isrc_v.at[j0]], buf0, sem0).wait()
            pltpu.sync_copy(buf0, acc.at[idst_v.at[j0]], add=True)

            @pl.when(j0 + 2 < SUP)
            def _():
                pltpu.async_copy(h2_hbm.at[isrc_v.at[j0 + 2]], buf0, sem0)

            pltpu.make_async_copy(h2_hbm.at[isrc_v.at[j0 + 1]], buf1, sem1).wait()
            pltpu.sync_copy(buf1, acc.at[idst_v.at[j0 + 1]], add=True)
            return ()

        lax.fori_loop(0, SUP // 2, body, (), unroll=False)
        return ()

    lax.fori_loop(0, BASE_CHUNKS // SUP, outer, (), unroll=False)

    plsc.subcore_barrier()
    pltpu.sync_copy(
        acc.at[pl.ds(sid * STRIPE, STRIPE)],
        out_hbm.at[cid, pl.ds(sid * STRIPE, STRIPE)],
    )


def _agg_call(src2, dst2, h2, zeros2d):
    return pl.kernel(
        _agg_body,
        out_type=jax.ShapeDtypeStruct((NC, NPAD, D), jnp.float32),
        mesh=_mesh(),
        scratch_types=[
            pltpu.VMEM((SUP, CHUNK), jnp.int32),
            pltpu.VMEM((SUP, CHUNK), jnp.int32),
            pltpu.VMEM((CHUNK, D), jnp.float32),
            pltpu.VMEM((CHUNK, D), jnp.float32),
            pltpu.SemaphoreType.DMA,
            pltpu.SemaphoreType.DMA,
            pltpu.VMEM_SHARED((NPAD, D), jnp.float32),
        ],
    )(src2, dst2, h2, zeros2d)


def _bn_body(aggp_ref, h2_ref, dinv_ref, b_ref, gamma_ref, beta_ref, out_ref):
    s = aggp_ref[0, :N_NODES, :] + aggp_ref[1, :N_NODES, :] + h2_ref[...]
    pre = s * dinv_ref[...][:, None] + b_ref[...]
    mean = jnp.mean(pre, axis=0)
    var = jnp.mean((pre - mean[None, :]) ** 2, axis=0)
    scaled = (pre - mean[None, :]) * lax.rsqrt(var + 1e-5)[None, :]
    out_ref[...] = jnp.maximum(scaled * gamma_ref[...] + beta_ref[...], 0.0)


def _bn_call(aggp, h2, dinv, b, gamma, beta):
    return pl.pallas_call(
        _bn_body,
        out_shape=jax.ShapeDtypeStruct((N_NODES, D), jnp.float32),
    )(aggp, h2, dinv, b, gamma, beta)


def kernel(x, edge_index, W, b, gamma, beta):
    ei = edge_index.astype(jnp.int32)
    npad = E_PAD - N_EDGES
    pad_src = jnp.arange(npad, dtype=jnp.int32) % N_NODES
    src2 = jnp.concatenate([ei[0], pad_src]).reshape(NCHUNKS, CHUNK)
    pad_dst = N_NODES + jnp.arange(npad, dtype=jnp.int32) % (NPAD - N_NODES)
    dst2 = jnp.concatenate([ei[1], pad_dst]).reshape(NCHUNKS, CHUNK)
    zeros1d = jnp.zeros((STRIPE,), jnp.float32)
    zeros2d = jnp.zeros((STRIPE, D), jnp.float32)

    degp = _deg_call(dst2, zeros1d)
    h2, dinv = _mm_call(x, W, degp)
    aggp = _agg_call(src2, dst2, h2, zeros2d)
    return _bn_call(aggp, h2, dinv, b, gamma, beta)


# speedup vs baseline: 41.4296x; 1.0392x over previous
import functools

import jax
import jax.numpy as jnp
from jax import lax
from jax.experimental import pallas as pl
from jax.experimental.pallas import tpu as pltpu
import jax.experimental.pallas.tpu_sc as plsc

N_NODES = 10000
N_EDGES = 320000
D = 128
NC = 2
NS = 16
NW = NC * NS
CHUNK = 128
STRIPE = 640
NPAD = STRIPE * 16
BASE_CHUNKS = 80
NCHUNKS = BASE_CHUNKS * NW
E_PAD = NCHUNKS * CHUNK
DUMMY_DST = NPAD - 8
SUP = 8


def _mesh():
    return plsc.VectorSubcoreMesh(
        core_axis_name="c", subcore_axis_name="s", num_cores=NC, num_subcores=NS
    )


def _worker_id():
    return lax.axis_index("s") + NS * lax.axis_index("c")


def _deg_body(dst2_hbm, zeros_hbm, out_hbm, idx_v, ones_v, hsem, acc):
    cid = lax.axis_index("c")
    sid = lax.axis_index("s")
    wid = _worker_id()

    pltpu.sync_copy(zeros_hbm, acc.at[pl.ds(sid * STRIPE, STRIPE)])
    for j in range(CHUNK // 16):
        ones_v[pl.ds(j * 16, 16)] = jnp.full((16,), 1.0, jnp.float32)
    plsc.subcore_barrier()

    pltpu.sync_copy(dst2_hbm.at[pl.ds(wid * BASE_CHUNKS, BASE_CHUNKS)], idx_v)

    def body(j, _):
        pltpu.async_copy(ones_v, acc.at[idx_v.at[j]], hsem, add=True)
        return ()

    lax.fori_loop(0, BASE_CHUNKS, body, (), unroll=False)

    def drain(j, _):
        pltpu.make_async_copy(ones_v, acc.at[idx_v.at[j]], hsem).wait()
        return ()

    lax.fori_loop(0, BASE_CHUNKS, drain, (), unroll=False)

    plsc.subcore_barrier()
    pltpu.sync_copy(
        acc.at[pl.ds(sid * STRIPE, STRIPE)],
        out_hbm.at[cid, pl.ds(sid * STRIPE, STRIPE)],
    )


def _deg_call(dst2, zeros1d):
    return pl.kernel(
        _deg_body,
        out_type=jax.ShapeDtypeStruct((NC, NPAD), jnp.float32),
        mesh=_mesh(),
        scratch_types=[
            pltpu.VMEM((BASE_CHUNKS, CHUNK), jnp.int32),
            pltpu.VMEM((CHUNK,), jnp.float32),
            pltpu.SemaphoreType.DMA,
            pltpu.VMEM_SHARED((NPAD,), jnp.float32),
        ],
    )(dst2, zeros1d)


def _mm_body(x_ref, w_ref, degp_ref, h2_ref, dinv_ref):
    deg = degp_ref[0, :N_NODES] + degp_ref[1, :N_NODES] + 1.0
    dinv = lax.rsqrt(deg)
    dinv_ref[...] = dinv
    h = jnp.dot(x_ref[...], w_ref[...], preferred_element_type=jnp.float32)
    h2_ref[...] = h * dinv[:, None]


def _mm_call(x, w, degp):
    return pl.pallas_call(
        _mm_body,
        out_shape=(
            jax.ShapeDtypeStruct((N_NODES, D), jnp.float32),
            jax.ShapeDtypeStruct((N_NODES,), jnp.float32),
        ),
    )(x, w, degp)


def _agg_body(src2_hbm, dst2_hbm, h2_hbm, zeros_hbm, out_hbm,
              isrc_v, idst_v, buf0, buf1, sem0, sem1, acc):
    cid = lax.axis_index("c")
    sid = lax.axis_index("s")
    wid = _worker_id()

    @pl.when(cid == 0)
    def _():
        @pl.when(sid < NS - 1)
        def _():
            pltpu.sync_copy(h2_hbm.at[pl.ds(sid * STRIPE, STRIPE)],
                            acc.at[pl.ds(sid * STRIPE, STRIPE)])

        @pl.when(sid == NS - 1)
        def _():
            pltpu.sync_copy(h2_hbm.at[pl.ds((NS - 1) * STRIPE, N_NODES - (NS - 1) * STRIPE)],
                            acc.at[pl.ds((NS - 1) * STRIPE, N_NODES - (NS - 1) * STRIPE)])
            pltpu.sync_copy(zeros_hbm.at[pl.ds(0, NPAD - N_NODES)],
                            acc.at[pl.ds(N_NODES, NPAD - N_NODES)])

    @pl.when(cid != 0)
    def _():
        pltpu.sync_copy(zeros_hbm, acc.at[pl.ds(sid * STRIPE, STRIPE)])

    plsc.subcore_barrier()

    def outer(t, _):
        base = wid * BASE_CHUNKS + t * SUP
        pltpu.sync_copy(src2_hbm.at[pl.ds(base, SUP)], isrc_v)
        pltpu.sync_copy(dst2_hbm.at[pl.ds(base, SUP)], idst_v)
        pltpu.async_copy(h2_hbm.at[isrc_v.at[0]], buf0, sem0)

        def body(k, _):
            j0 = 2 * k
            pltpu.async_copy(h2_hbm.at[isrc_v.at[j0 + 1]], buf1, sem1)
            pltpu.make_async_copy(h2_hbm.at[isrc_v.at[j0]], buf0, sem0).wait()
            pltpu.sync_copy(buf0, acc.at[idst_v.at[j0]], add=True)

            @pl.when(j0 + 2 < SUP)
            def _():
                pltpu.async_copy(h2_hbm.at[isrc_v.at[j0 + 2]], buf0, sem0)

            pltpu.make_async_copy(h2_hbm.at[isrc_v.at[j0 + 1]], buf1, sem1).wait()
            pltpu.sync_copy(buf1, acc.at[idst_v.at[j0 + 1]], add=True)
            return ()

        lax.fori_loop(0, SUP // 2, body, (), unroll=False)
        return ()

    lax.fori_loop(0, BASE_CHUNKS // SUP, outer, (), unroll=False)

    plsc.subcore_barrier()
    pltpu.sync_copy(
        acc.at[pl.ds(sid * STRIPE, STRIPE)],
        out_hbm.at[cid, pl.ds(sid * STRIPE, STRIPE)],
    )


def _agg_call(src2, dst2, h2, zeros2d):
    return pl.kernel(
        _agg_body,
        out_type=jax.ShapeDtypeStruct((NC, NPAD, D), jnp.float32),
        mesh=_mesh(),
        scratch_types=[
            pltpu.VMEM((SUP, CHUNK), jnp.int32),
            pltpu.VMEM((SUP, CHUNK), jnp.int32),
            pltpu.VMEM((CHUNK, D), jnp.float32),
            pltpu.VMEM((CHUNK, D), jnp.float32),
            pltpu.SemaphoreType.DMA,
            pltpu.SemaphoreType.DMA,
            pltpu.VMEM_SHARED((NPAD, D), jnp.float32),
        ],
    )(src2, dst2, h2, zeros2d)


def _bn_body(aggp_ref, dinv_ref, b_ref, gamma_ref, beta_ref, out_ref):
    s = aggp_ref[0, :N_NODES, :] + aggp_ref[1, :N_NODES, :]
    pre = s * dinv_ref[...][:, None] + b_ref[...]
    mean = jnp.mean(pre, axis=0)
    var = jnp.mean((pre - mean[None, :]) ** 2, axis=0)
    scaled = (pre - mean[None, :]) * lax.rsqrt(var + 1e-5)[None, :]
    out_ref[...] = jnp.maximum(scaled * gamma_ref[...] + beta_ref[...], 0.0)


def _bn_call(aggp, dinv, b, gamma, beta):
    return pl.pallas_call(
        _bn_body,
        out_shape=jax.ShapeDtypeStruct((N_NODES, D), jnp.float32),
    )(aggp, dinv, b, gamma, beta)


def kernel(x, edge_index, W, b, gamma, beta):
    ei = edge_index.astype(jnp.int32)
    npad = E_PAD - N_EDGES
    pad_src = jnp.arange(npad, dtype=jnp.int32) % N_NODES
    src2 = jnp.concatenate([ei[0], pad_src]).reshape(NCHUNKS, CHUNK)
    pad_dst = N_NODES + jnp.arange(npad, dtype=jnp.int32) % (NPAD - N_NODES)
    dst2 = jnp.concatenate([ei[1], pad_dst]).reshape(NCHUNKS, CHUNK)
    zeros1d = jnp.zeros((STRIPE,), jnp.float32)
    zeros2d = jnp.zeros((STRIPE, D), jnp.float32)

    degp = _deg_call(dst2, zeros1d)
    h2, dinv = _mm_call(x, W, degp)
    aggp = _agg_call(src2, dst2, h2, zeros2d)
    return _bn_call(aggp, dinv, b, gamma, beta)


# speedup vs baseline: 42.8580x; 1.0345x over previous
import functools

import jax
import jax.numpy as jnp
from jax import lax
from jax.experimental import pallas as pl
from jax.experimental.pallas import tpu as pltpu
import jax.experimental.pallas.tpu_sc as plsc

N_NODES = 10000
N_EDGES = 320000
D = 128
NC = 2
NS = 16
NW = NC * NS
CHUNK = 128
STRIPE = 640
NPAD = STRIPE * 16
BASE_CHUNKS = 80
NCHUNKS = BASE_CHUNKS * NW
E_PAD = NCHUNKS * CHUNK
DUMMY_DST = NPAD - 8
SUP = 8


def _mesh():
    return plsc.VectorSubcoreMesh(
        core_axis_name="c", subcore_axis_name="s", num_cores=NC, num_subcores=NS
    )


def _worker_id():
    return lax.axis_index("s") + NS * lax.axis_index("c")


def _deg_body(pei_hbm, zeros_hbm, out_hbm, idx_v, ones_v, hsem, acc):
    cid = lax.axis_index("c")
    sid = lax.axis_index("s")
    wid = _worker_id()

    pltpu.sync_copy(zeros_hbm, acc.at[pl.ds(sid * STRIPE, STRIPE)])
    for j in range(CHUNK // 16):
        ones_v[pl.ds(j * 16, 16)] = jnp.full((16,), 1.0, jnp.float32)
    plsc.subcore_barrier()

    pltpu.sync_copy(pei_hbm.at[1, pl.ds(wid * BASE_CHUNKS, BASE_CHUNKS)], idx_v)

    def body(j, _):
        pltpu.async_copy(ones_v, acc.at[idx_v.at[j]], hsem, add=True)
        return ()

    lax.fori_loop(0, BASE_CHUNKS, body, (), unroll=False)

    def drain(j, _):
        pltpu.make_async_copy(ones_v, acc.at[idx_v.at[j]], hsem).wait()
        return ()

    lax.fori_loop(0, BASE_CHUNKS, drain, (), unroll=False)

    plsc.subcore_barrier()
    pltpu.sync_copy(
        acc.at[pl.ds(sid * STRIPE, STRIPE)],
        out_hbm.at[cid, pl.ds(sid * STRIPE, STRIPE)],
    )


def _deg_call(pei, zeros1d):
    return pl.kernel(
        _deg_body,
        out_type=jax.ShapeDtypeStruct((NC, NPAD), jnp.float32),
        mesh=_mesh(),
        scratch_types=[
            pltpu.VMEM((BASE_CHUNKS, CHUNK), jnp.int32),
            pltpu.VMEM((CHUNK,), jnp.float32),
            pltpu.SemaphoreType.DMA,
            pltpu.VMEM_SHARED((NPAD,), jnp.float32),
        ],
    )(pei, zeros1d)


def _mm_body(x_ref, w_ref, degp_ref, h2_ref, dinv_ref):
    deg = degp_ref[0, :N_NODES] + degp_ref[1, :N_NODES] + 1.0
    dinv = lax.rsqrt(deg)
    dinv_ref[...] = dinv
    h = jnp.dot(x_ref[...], w_ref[...], preferred_element_type=jnp.float32)
    h2_ref[...] = h * dinv[:, None]


def _mm_call(x, w, degp):
    return pl.pallas_call(
        _mm_body,
        out_shape=(
            jax.ShapeDtypeStruct((N_NODES, D), jnp.float32),
            jax.ShapeDtypeStruct((N_NODES,), jnp.float32),
        ),
    )(x, w, degp)


def _agg_body(pei_hbm, h2_hbm, zeros_hbm, out_hbm,
              isrc_v, idst_v, buf0, buf1, sem0, sem1, acc):
    cid = lax.axis_index("c")
    sid = lax.axis_index("s")
    wid = _worker_id()

    @pl.when(cid == 0)
    def _():
        @pl.when(sid < NS - 1)
        def _():
            pltpu.sync_copy(h2_hbm.at[pl.ds(sid * STRIPE, STRIPE)],
                            acc.at[pl.ds(sid * STRIPE, STRIPE)])

        @pl.when(sid == NS - 1)
        def _():
            pltpu.sync_copy(h2_hbm.at[pl.ds((NS - 1) * STRIPE, N_NODES - (NS - 1) * STRIPE)],
                            acc.at[pl.ds((NS - 1) * STRIPE, N_NODES - (NS - 1) * STRIPE)])
            pltpu.sync_copy(zeros_hbm.at[pl.ds(0, NPAD - N_NODES)],
                            acc.at[pl.ds(N_NODES, NPAD - N_NODES)])

    @pl.when(cid != 0)
    def _():
        pltpu.sync_copy(zeros_hbm, acc.at[pl.ds(sid * STRIPE, STRIPE)])

    plsc.subcore_barrier()

    def outer(t, _):
        base = wid * BASE_CHUNKS + t * SUP
        pltpu.sync_copy(pei_hbm.at[0, pl.ds(base, SUP)], isrc_v)
        pltpu.sync_copy(pei_hbm.at[1, pl.ds(base, SUP)], idst_v)
        pltpu.async_copy(h2_hbm.at[isrc_v.at[0]], buf0, sem0)

        def body(k, _):
            j0 = 2 * k
            pltpu.async_copy(h2_hbm.at[isrc_v.at[j0 + 1]], buf1, sem1)
            pltpu.make_async_copy(h2_hbm.at[isrc_v.at[j0]], buf0, sem0).wait()
            pltpu.sync_copy(buf0, acc.at[idst_v.at[j0]], add=True)

            @pl.when(j0 + 2 < SUP)
            def _():
                pltpu.async_copy(h2_hbm.at[isrc_v.at[j0 + 2]], buf0, sem0)

            pltpu.make_async_copy(h2_hbm.at[isrc_v.at[j0 + 1]], buf1, sem1).wait()
            pltpu.sync_copy(buf1, acc.at[idst_v.at[j0 + 1]], add=True)
            return ()

        lax.fori_loop(0, SUP // 2, body, (), unroll=False)
        return ()

    lax.fori_loop(0, BASE_CHUNKS // SUP, outer, (), unroll=False)

    plsc.subcore_barrier()
    pltpu.sync_copy(
        acc.at[pl.ds(sid * STRIPE, STRIPE)],
        out_hbm.at[cid, pl.ds(sid * STRIPE, STRIPE)],
    )


def _agg_call(pei, h2, zeros2d):
    return pl.kernel(
        _agg_body,
        out_type=jax.ShapeDtypeStruct((NC, NPAD, D), jnp.float32),
        mesh=_mesh(),
        scratch_types=[
            pltpu.VMEM((SUP, CHUNK), jnp.int32),
            pltpu.VMEM((SUP, CHUNK), jnp.int32),
            pltpu.VMEM((CHUNK, D), jnp.float32),
            pltpu.VMEM((CHUNK, D), jnp.float32),
            pltpu.SemaphoreType.DMA,
            pltpu.SemaphoreType.DMA,
            pltpu.VMEM_SHARED((NPAD, D), jnp.float32),
        ],
    )(pei, h2, zeros2d)


def _bn_body(aggp_ref, dinv_ref, b_ref, gamma_ref, beta_ref, out_ref):
    s = aggp_ref[0, :N_NODES, :] + aggp_ref[1, :N_NODES, :]
    pre = s * dinv_ref[...][:, None] + b_ref[...]
    mean = jnp.mean(pre, axis=0)
    var = jnp.mean((pre - mean[None, :]) ** 2, axis=0)
    scaled = (pre - mean[None, :]) * lax.rsqrt(var + 1e-5)[None, :]
    out_ref[...] = jnp.maximum(scaled * gamma_ref[...] + beta_ref[...], 0.0)


def _bn_call(aggp, dinv, b, gamma, beta):
    return pl.pallas_call(
        _bn_body,
        out_shape=jax.ShapeDtypeStruct((N_NODES, D), jnp.float32),
    )(aggp, dinv, b, gamma, beta)


def kernel(x, edge_index, W, b, gamma, beta):
    ei = edge_index.astype(jnp.int32)
    npad = E_PAD - N_EDGES
    pad_src = jnp.arange(npad, dtype=jnp.int32) % N_NODES
    pad_dst = N_NODES + jnp.arange(npad, dtype=jnp.int32) % (NPAD - N_NODES)
    pei = jnp.concatenate(
        [ei, jnp.stack([pad_src, pad_dst])], axis=1).reshape(2, NCHUNKS, CHUNK)
    zeros1d = jnp.zeros((STRIPE,), jnp.float32)
    zeros2d = jnp.zeros((STRIPE, D), jnp.float32)

    degp = _deg_call(pei, zeros1d)
    h2, dinv = _mm_call(x, W, degp)
    aggp = _agg_call(pei, h2, zeros2d)
    return _bn_call(aggp, dinv, b, gamma, beta)


# speedup vs baseline: 43.1058x; 1.0058x over previous
import functools

import jax
import jax.numpy as jnp
from jax import lax
from jax.experimental import pallas as pl
from jax.experimental.pallas import tpu as pltpu
import jax.experimental.pallas.tpu_sc as plsc

N_NODES = 10000
N_EDGES = 320000
D = 128
NC = 2
NS = 16
NW = NC * NS
CHUNK = 128
STRIPE = 640
NPAD = STRIPE * 16
BASE_CHUNKS = 80
NCHUNKS = BASE_CHUNKS * NW
E_PAD = NCHUNKS * CHUNK
DUMMY_DST = NPAD - 8
SUP = 8


def _mesh():
    return plsc.VectorSubcoreMesh(
        core_axis_name="c", subcore_axis_name="s", num_cores=NC, num_subcores=NS
    )


def _worker_id():
    return lax.axis_index("s") + NS * lax.axis_index("c")


def _deg_body(pei_hbm, zeros_hbm, out_hbm, idx_v, ones_v, hsem, acc):
    cid = lax.axis_index("c")
    sid = lax.axis_index("s")
    wid = _worker_id()

    pltpu.sync_copy(zeros_hbm, acc.at[pl.ds(sid * STRIPE, STRIPE)])
    for j in range(CHUNK // 16):
        ones_v[pl.ds(j * 16, 16)] = jnp.full((16,), 1.0, jnp.float32)
    plsc.subcore_barrier()

    pltpu.sync_copy(pei_hbm.at[1, pl.ds(wid * BASE_CHUNKS, BASE_CHUNKS)], idx_v)

    def body(j, _):
        pltpu.async_copy(ones_v, acc.at[idx_v.at[j]], hsem, add=True)
        return ()

    lax.fori_loop(0, BASE_CHUNKS, body, (), unroll=False)

    def drain(j, _):
        pltpu.make_async_copy(ones_v, acc.at[idx_v.at[j]], hsem).wait()
        return ()

    lax.fori_loop(0, BASE_CHUNKS, drain, (), unroll=False)

    plsc.subcore_barrier()
    pltpu.sync_copy(
        acc.at[pl.ds(sid * STRIPE, STRIPE)],
        out_hbm.at[cid, pl.ds(sid * STRIPE, STRIPE)],
    )


def _deg_call(pei, zeros1d):
    return pl.kernel(
        _deg_body,
        out_type=jax.ShapeDtypeStruct((NC, NPAD), jnp.float32),
        mesh=_mesh(),
        scratch_types=[
            pltpu.VMEM((BASE_CHUNKS, CHUNK), jnp.int32),
            pltpu.VMEM((CHUNK,), jnp.float32),
            pltpu.SemaphoreType.DMA,
            pltpu.VMEM_SHARED((NPAD,), jnp.float32),
        ],
    )(pei, zeros1d)


def _mm_body(x_ref, w_ref, degp_ref, h2_ref, dinv_ref):
    deg = degp_ref[0, :N_NODES] + degp_ref[1, :N_NODES] + 1.0
    dinv = lax.rsqrt(deg)
    dinv_ref[...] = dinv
    h = jnp.dot(x_ref[...], w_ref[...], preferred_element_type=jnp.float32)
    h2_ref[...] = h * dinv[:, None]


def _mm_call(x, w, degp):
    return pl.pallas_call(
        _mm_body,
        out_shape=(
            jax.ShapeDtypeStruct((N_NODES, D), jnp.float32),
            jax.ShapeDtypeStruct((N_NODES,), jnp.float32),
        ),
    )(x, w, degp)


def _agg_body(pei_hbm, h2_hbm, zeros_hbm, out_hbm,
              isrc_v, idst_v, buf0, buf1, sem0, sem1, ssem0, ssem1, acc):
    cid = lax.axis_index("c")
    sid = lax.axis_index("s")
    wid = _worker_id()

    @pl.when(cid == 0)
    def _():
        @pl.when(sid < NS - 1)
        def _():
            pltpu.sync_copy(h2_hbm.at[pl.ds(sid * STRIPE, STRIPE)],
                            acc.at[pl.ds(sid * STRIPE, STRIPE)])

        @pl.when(sid == NS - 1)
        def _():
            pltpu.sync_copy(h2_hbm.at[pl.ds((NS - 1) * STRIPE, N_NODES - (NS - 1) * STRIPE)],
                            acc.at[pl.ds((NS - 1) * STRIPE, N_NODES - (NS - 1) * STRIPE)])
            pltpu.sync_copy(zeros_hbm.at[pl.ds(0, NPAD - N_NODES)],
                            acc.at[pl.ds(N_NODES, NPAD - N_NODES)])

    @pl.when(cid != 0)
    def _():
        pltpu.sync_copy(zeros_hbm, acc.at[pl.ds(sid * STRIPE, STRIPE)])

    plsc.subcore_barrier()

    def outer(t, _):
        base = wid * BASE_CHUNKS + t * SUP
        pltpu.sync_copy(pei_hbm.at[0, pl.ds(base, SUP)], isrc_v)
        pltpu.sync_copy(pei_hbm.at[1, pl.ds(base, SUP)], idst_v)
        pltpu.async_copy(h2_hbm.at[isrc_v.at[0]], buf0, sem0)

        def body(k, _):
            j0 = 2 * k
            pltpu.make_async_copy(h2_hbm.at[isrc_v.at[j0]], buf0, sem0).wait()
            pltpu.async_copy(buf0, acc.at[idst_v.at[j0]], ssem0, add=True)
            pltpu.async_copy(h2_hbm.at[isrc_v.at[j0 + 1]], buf1, sem1)
            pltpu.make_async_copy(buf0, acc.at[idst_v.at[j0]], ssem0).wait()

            @pl.when(j0 + 2 < SUP)
            def _():
                pltpu.async_copy(h2_hbm.at[isrc_v.at[j0 + 2]], buf0, sem0)

            pltpu.make_async_copy(h2_hbm.at[isrc_v.at[j0 + 1]], buf1, sem1).wait()
            pltpu.async_copy(buf1, acc.at[idst_v.at[j0 + 1]], ssem1, add=True)
            pltpu.make_async_copy(buf1, acc.at[idst_v.at[j0 + 1]], ssem1).wait()
            return ()

        lax.fori_loop(0, SUP // 2, body, (), unroll=False)
        return ()

    lax.fori_loop(0, BASE_CHUNKS // SUP, outer, (), unroll=False)

    plsc.subcore_barrier()
    pltpu.sync_copy(
        acc.at[pl.ds(sid * STRIPE, STRIPE)],
        out_hbm.at[cid, pl.ds(sid * STRIPE, STRIPE)],
    )


def _agg_call(pei, h2, zeros2d):
    return pl.kernel(
        _agg_body,
        out_type=jax.ShapeDtypeStruct((NC, NPAD, D), jnp.float32),
        mesh=_mesh(),
        scratch_types=[
            pltpu.VMEM((SUP, CHUNK), jnp.int32),
            pltpu.VMEM((SUP, CHUNK), jnp.int32),
            pltpu.VMEM((CHUNK, D), jnp.float32),
            pltpu.VMEM((CHUNK, D), jnp.float32),
            pltpu.SemaphoreType.DMA,
            pltpu.SemaphoreType.DMA,
            pltpu.SemaphoreType.DMA,
            pltpu.SemaphoreType.DMA,
            pltpu.VMEM_SHARED((NPAD, D), jnp.float32),
        ],
    )(pei, h2, zeros2d)


def _bn_body(aggp_ref, dinv_ref, b_ref, gamma_ref, beta_ref, out_ref):
    s = aggp_ref[0, :N_NODES, :] + aggp_ref[1, :N_NODES, :]
    pre = s * dinv_ref[...][:, None] + b_ref[...]
    mean = jnp.mean(pre, axis=0)
    var = jnp.mean((pre - mean[None, :]) ** 2, axis=0)
    scaled = (pre - mean[None, :]) * lax.rsqrt(var + 1e-5)[None, :]
    out_ref[...] = jnp.maximum(scaled * gamma_ref[...] + beta_ref[...], 0.0)


def _bn_call(aggp, dinv, b, gamma, beta):
    return pl.pallas_call(
        _bn_body,
        out_shape=jax.ShapeDtypeStruct((N_NODES, D), jnp.float32),
    )(aggp, dinv, b, gamma, beta)


def kernel(x, edge_index, W, b, gamma, beta):
    ei = edge_index.astype(jnp.int32)
    npad = E_PAD - N_EDGES
    pad_src = jnp.arange(npad, dtype=jnp.int32) % N_NODES
    pad_dst = N_NODES + jnp.arange(npad, dtype=jnp.int32) % (NPAD - N_NODES)
    pei = jnp.concatenate(
        [ei, jnp.stack([pad_src, pad_dst])], axis=1).reshape(2, NCHUNKS, CHUNK)
    zeros1d = jnp.zeros((STRIPE,), jnp.float32)
    zeros2d = jnp.zeros((STRIPE, D), jnp.float32)

    degp = _deg_call(pei, zeros1d)
    h2, dinv = _mm_call(x, W, degp)
    aggp = _agg_call(pei, h2, zeros2d)
    return _bn_call(aggp, dinv, b, gamma, beta)


# speedup vs baseline: 45.8707x; 1.0641x over previous
import functools

import jax
import jax.numpy as jnp
from jax import lax
from jax.experimental import pallas as pl
from jax.experimental.pallas import tpu as pltpu
import jax.experimental.pallas.tpu_sc as plsc

N_NODES = 10000
N_EDGES = 320000
D = 128
NC = 2
NS = 16
NW = NC * NS
CHUNK = 64
STRIPE = 640
NPAD = STRIPE * 16
BASE_CHUNKS = 160
NCHUNKS = BASE_CHUNKS * NW
E_PAD = NCHUNKS * CHUNK
DUMMY_DST = NPAD - 8
SUP = 16
NB = 4


def _mesh():
    return plsc.VectorSubcoreMesh(
        core_axis_name="c", subcore_axis_name="s", num_cores=NC, num_subcores=NS
    )


def _worker_id():
    return lax.axis_index("s") + NS * lax.axis_index("c")


def _deg_body(pei_hbm, zeros_hbm, out_hbm, idx_v, ones_v, hsem, acc):
    cid = lax.axis_index("c")
    sid = lax.axis_index("s")
    wid = _worker_id()

    pltpu.sync_copy(zeros_hbm, acc.at[pl.ds(sid * STRIPE, STRIPE)])
    for j in range(CHUNK // 16):
        ones_v[pl.ds(j * 16, 16)] = jnp.full((16,), 1.0, jnp.float32)
    plsc.subcore_barrier()

    pltpu.sync_copy(pei_hbm.at[1, pl.ds(wid * BASE_CHUNKS, BASE_CHUNKS)], idx_v)

    def body(j, _):
        pltpu.async_copy(ones_v, acc.at[idx_v.at[j]], hsem, add=True)
        return ()

    lax.fori_loop(0, BASE_CHUNKS, body, (), unroll=False)

    def drain(j, _):
        pltpu.make_async_copy(ones_v, acc.at[idx_v.at[j]], hsem).wait()
        return ()

    lax.fori_loop(0, BASE_CHUNKS, drain, (), unroll=False)

    plsc.subcore_barrier()
    pltpu.sync_copy(
        acc.at[pl.ds(sid * STRIPE, STRIPE)],
        out_hbm.at[cid, pl.ds(sid * STRIPE, STRIPE)],
    )


def _deg_call(pei, zeros1d):
    return pl.kernel(
        _deg_body,
        out_type=jax.ShapeDtypeStruct((NC, NPAD), jnp.float32),
        mesh=_mesh(),
        scratch_types=[
            pltpu.VMEM((BASE_CHUNKS, CHUNK), jnp.int32),
            pltpu.VMEM((CHUNK,), jnp.float32),
            pltpu.SemaphoreType.DMA,
            pltpu.VMEM_SHARED((NPAD,), jnp.float32),
        ],
    )(pei, zeros1d)


def _mm_body(x_ref, w_ref, degp_ref, h2_ref, dinv_ref):
    deg = degp_ref[0, :N_NODES] + degp_ref[1, :N_NODES] + 1.0
    dinv = lax.rsqrt(deg)
    dinv_ref[...] = dinv
    h = jnp.dot(x_ref[...], w_ref[...], preferred_element_type=jnp.float32)
    h2_ref[...] = h * dinv[:, None]


def _mm_call(x, w, degp):
    return pl.pallas_call(
        _mm_body,
        out_shape=(
            jax.ShapeDtypeStruct((N_NODES, D), jnp.float32),
            jax.ShapeDtypeStruct((N_NODES,), jnp.float32),
        ),
    )(x, w, degp)


def _agg_body(pei_hbm, h2_hbm, zeros_hbm, out_hbm,
              idx_v, b0, b1, b2, b3, g0, g1, g2, g3, s0, s1, s2, s3, acc):
    cid = lax.axis_index("c")
    sid = lax.axis_index("s")
    wid = _worker_id()
    bufs = [b0, b1, b2, b3]
    gsem = [g0, g1, g2, g3]
    ssem = [s0, s1, s2, s3]

    @pl.when(cid == 0)
    def _():
        @pl.when(sid < NS - 1)
        def _():
            pltpu.sync_copy(h2_hbm.at[pl.ds(sid * STRIPE, STRIPE)],
                            acc.at[pl.ds(sid * STRIPE, STRIPE)])

        @pl.when(sid == NS - 1)
        def _():
            pltpu.sync_copy(h2_hbm.at[pl.ds((NS - 1) * STRIPE, N_NODES - (NS - 1) * STRIPE)],
                            acc.at[pl.ds((NS - 1) * STRIPE, N_NODES - (NS - 1) * STRIPE)])
            pltpu.sync_copy(zeros_hbm.at[pl.ds(0, NPAD - N_NODES)],
                            acc.at[pl.ds(N_NODES, NPAD - N_NODES)])

    @pl.when(cid != 0)
    def _():
        pltpu.sync_copy(zeros_hbm, acc.at[pl.ds(sid * STRIPE, STRIPE)])

    plsc.subcore_barrier()

    def outer(t, _):
        base = wid * BASE_CHUNKS + t * SUP
        pltpu.sync_copy(pei_hbm.at[:, pl.ds(base, SUP)], idx_v)
        for j in range(NB):
            pltpu.async_copy(h2_hbm.at[idx_v.at[0, j]], bufs[j], gsem[j])
        for j in range(SUP):
            b = j % NB
            pltpu.make_async_copy(h2_hbm.at[idx_v.at[0, j]], bufs[b], gsem[b]).wait()
            pltpu.async_copy(bufs[b], acc.at[idx_v.at[1, j]], ssem[b], add=True)
            if j + NB < SUP:
                pltpu.make_async_copy(bufs[b], acc.at[idx_v.at[1, j]], ssem[b]).wait()
                pltpu.async_copy(h2_hbm.at[idx_v.at[0, j + NB]], bufs[b], gsem[b])
        for j in range(SUP - NB, SUP):
            b = j % NB
            pltpu.make_async_copy(bufs[b], acc.at[idx_v.at[1, j]], ssem[b]).wait()
        return ()

    lax.fori_loop(0, BASE_CHUNKS // SUP, outer, (), unroll=False)

    plsc.subcore_barrier()
    pltpu.sync_copy(
        acc.at[pl.ds(sid * STRIPE, STRIPE)],
        out_hbm.at[cid, pl.ds(sid * STRIPE, STRIPE)],
    )


def _agg_call(pei, h2, zeros2d):
    return pl.kernel(
        _agg_body,
        out_type=jax.ShapeDtypeStruct((NC, NPAD, D), jnp.float32),
        mesh=_mesh(),
        scratch_types=[
            pltpu.VMEM((2, SUP, CHUNK), jnp.int32),
            pltpu.VMEM((CHUNK, D), jnp.float32),
            pltpu.VMEM((CHUNK, D), jnp.float32),
            pltpu.VMEM((CHUNK, D), jnp.float32),
            pltpu.VMEM((CHUNK, D), jnp.float32),
            pltpu.SemaphoreType.DMA,
            pltpu.SemaphoreType.DMA,
            pltpu.SemaphoreType.DMA,
            pltpu.SemaphoreType.DMA,
            pltpu.SemaphoreType.DMA,
            pltpu.SemaphoreType.DMA,
            pltpu.SemaphoreType.DMA,
            pltpu.SemaphoreType.DMA,
            pltpu.VMEM_SHARED((NPAD, D), jnp.float32),
        ],
    )(pei, h2, zeros2d)


def _bn_body(aggp_ref, dinv_ref, b_ref, gamma_ref, beta_ref, out_ref):
    s = aggp_ref[0, :N_NODES, :] + aggp_ref[1, :N_NODES, :]
    pre = s * dinv_ref[...][:, None] + b_ref[...]
    mean = jnp.mean(pre, axis=0)
    var = jnp.mean((pre - mean[None, :]) ** 2, axis=0)
    scaled = (pre - mean[None, :]) * lax.rsqrt(var + 1e-5)[None, :]
    out_ref[...] = jnp.maximum(scaled * gamma_ref[...] + beta_ref[...], 0.0)


def _bn_call(aggp, dinv, b, gamma, beta):
    return pl.pallas_call(
        _bn_body,
        out_shape=jax.ShapeDtypeStruct((N_NODES, D), jnp.float32),
    )(aggp, dinv, b, gamma, beta)


def kernel(x, edge_index, W, b, gamma, beta):
    ei = edge_index.astype(jnp.int32)
    npad = E_PAD - N_EDGES
    pad_src = jnp.arange(npad, dtype=jnp.int32) % N_NODES
    pad_dst = N_NODES + jnp.arange(npad, dtype=jnp.int32) % (NPAD - N_NODES)
    pei = jnp.concatenate(
        [ei, jnp.stack([pad_src, pad_dst])], axis=1).reshape(2, NCHUNKS, CHUNK)
    zeros1d = jnp.zeros((STRIPE,), jnp.float32)
    zeros2d = jnp.zeros((STRIPE, D), jnp.float32)

    degp = _deg_call(pei, zeros1d)
    h2, dinv = _mm_call(x, W, degp)
    aggp = _agg_call(pei, h2, zeros2d)
    return _bn_call(aggp, dinv, b, gamma, beta)
